# baseline (speedup 1.0000x reference)
import functools

import jax
import jax.numpy as jnp
from jax import lax
from jax.experimental import pallas as pl
from jax.experimental.pallas import tpu as pltpu
from jax.experimental.pallas import tpu_sc as plsc

N_NODES = 10000
N_EDGES = 320000
IN_DIM = 128
HID = 128
OUT_DIM = 64
NUM_GRAPHS = 64

NC, NS, L = 2, 16, 16
NW = NC * NS
K = 128
NCH = 80
EPT = K * NCH
NPAD = 10112
RPT = NPAD // NS
NCH_H = NCH // 2

_MESH = plsc.VectorSubcoreMesh(core_axis_name="c", subcore_axis_name="s",
                               num_cores=NC, num_subcores=NS)


def _deg_body(dst_hbm, out_hbm, idx_v, deg_v):
    c = lax.axis_index("c")
    s = lax.axis_index("s")
    wid = c * NS + s
    pltpu.sync_copy(dst_hbm.at[wid], idx_v)
    zero16 = jnp.zeros((L,), jnp.float32)
    ones16 = jnp.full((L,), 1.0, jnp.float32)

    def zbody(i, carry):
        deg_v[pl.ds(pl.multiple_of(i * L, L), L)] = zero16
        return carry

    lax.fori_loop(0, NPAD // L, zbody, 0)

    def body(j, carry):
        for ci in range(K // L):
            idx16 = idx_v[j, pl.ds(ci * L, L)]
            plsc.addupdate_scatter(deg_v, [idx16], ones16)
        return carry

    lax.fori_loop(0, NCH, body, 0)
    pltpu.sync_copy(deg_v, out_hbm.at[wid])


_deg_call = functools.partial(
    pl.kernel,
    out_type=jax.ShapeDtypeStruct((NW, NPAD), jnp.float32),
    mesh=_MESH,
    compiler_params=pltpu.CompilerParams(needs_layout_passes=False),
    scratch_types=[
        pltpu.VMEM((NCH, K), jnp.int32),
        pltpu.VMEM((NPAD,), jnp.float32),
    ],
)(_deg_body)


def _agg_body(src_hbm, dst_hbm, h_hbm, zeros_hbm, out_hbm,
              src_v, dst_v, rows_v, agg_sh, gsem0, gsem1, ssem0, ssem1):
    c = lax.axis_index("c")
    s = lax.axis_index("s")
    wid = c * NS + s
    r0 = pl.multiple_of(s * RPT, 8)
    pltpu.sync_copy(zeros_hbm.at[pl.ds(r0, RPT)], agg_sh.at[pl.ds(r0, RPT)])
    plsc.subcore_barrier()

    for half in range(2):
        pltpu.sync_copy(src_hbm.at[wid, pl.ds(half * NCH_H, NCH_H)], src_v)
        pltpu.sync_copy(dst_hbm.at[wid, pl.ds(half * NCH_H, NCH_H)], dst_v)
        pltpu.async_copy(h_hbm.at[src_v.at[0]], rows_v.at[0], gsem0)
        pltpu.async_copy(h_hbm.at[src_v.at[1]], rows_v.at[1], gsem1)

        def body(g, carry):
            j0 = g * 2
            j1 = j0 + 1
            pltpu.make_async_copy(h_hbm.at[src_v.at[j0]], rows_v.at[0], gsem0).wait()
            pltpu.async_copy(rows_v.at[0], agg_sh.at[dst_v.at[j0]], ssem0, add=True)
            pltpu.make_async_copy(h_hbm.at[src_v.at[j1]], rows_v.at[1], gsem1).wait()
            pltpu.async_copy(rows_v.at[1], agg_sh.at[dst_v.at[j1]], ssem1, add=True)
            pltpu.make_async_copy(rows_v.at[0], agg_sh.at[dst_v.at[j0]], ssem0).wait()

            @pl.when(j0 + 2 < NCH_H)
            def _():
                pltpu.async_copy(h_hbm.at[src_v.at[j0 + 2]], rows_v.at[0], gsem0)

            pltpu.make_async_copy(rows_v.at[1], agg_sh.at[dst_v.at[j1]], ssem1).wait()

            @pl.when(j1 + 2 < NCH_H)
            def _():
                pltpu.async_copy(h_hbm.at[src_v.at[j1 + 2]], rows_v.at[1], gsem1)

            return carry

        lax.fori_loop(0, NCH_H // 2, body, 0)
    plsc.subcore_barrier()
    pltpu.sync_copy(agg_sh.at[pl.ds(r0, RPT)], out_hbm.at[c, pl.ds(r0, RPT)])


_agg_call = functools.partial(
    pl.kernel,
    out_type=jax.ShapeDtypeStruct((NC, NPAD, HID), jnp.float32),
    mesh=_MESH,
    scratch_types=[
        pltpu.VMEM((NCH_H, K), jnp.int32),
        pltpu.VMEM((NCH_H, K), jnp.int32),
        pltpu.VMEM((2, K, HID), jnp.float32),
        pltpu.VMEM_SHARED((NPAD, HID), jnp.float32),
        pltpu.SemaphoreType.DMA,
        pltpu.SemaphoreType.DMA,
        pltpu.SemaphoreType.DMA,
        pltpu.SemaphoreType.DMA,
    ],
)(_agg_body)


_RB = 1000
_NBLK = N_NODES // _RB


def _dinv_from(deg_ref):
    d = jnp.sum(deg_ref[...], axis=1, keepdims=True) + 1.0
    return lax.rsqrt(d)


def _tc1_body(x_ref, w1_ref, deg_ref, out_ref):
    dinv = _dinv_from(deg_ref)
    h = jnp.dot(x_ref[...], w1_ref[...], preferred_element_type=jnp.float32)
    out_ref[...] = h * dinv


def _tc1(x, W1, deg2):
    return pl.pallas_call(
        _tc1_body,
        grid=(_NBLK,),
        in_specs=[
            pl.BlockSpec((_RB, IN_DIM), lambda i: (i, 0)),
            pl.BlockSpec((IN_DIM, HID), lambda i: (0, 0)),
            pl.BlockSpec((_RB, NW), lambda i: (i, 0)),
        ],
        out_specs=pl.BlockSpec((_RB, HID), lambda i: (i, 0)),
        out_shape=jax.ShapeDtypeStruct((N_NODES, HID), jnp.float32),
    )(x, W1, deg2)


def _tc2_body(p_ref, hs_ref, deg_ref, b1_ref, w2_ref, out_ref):
    dinv = _dinv_from(deg_ref)
    agg = dinv * (p_ref[0] + p_ref[1] + hs_ref[...]) + b1_ref[...]
    t = jnp.maximum(agg, 0.0)
    out_ref[...] = jnp.dot(t, w2_ref[...],
                           preferred_element_type=jnp.float32) * dinv


def _tc2(P, hs1, deg2, b1, W2):
    return pl.pallas_call(
        _tc2_body,
        grid=(_NBLK,),
        in_specs=[
            pl.BlockSpec((NC, _RB, HID), lambda i: (0, i, 0)),
            pl.BlockSpec((_RB, HID), lambda i: (i, 0)),
            pl.BlockSpec((_RB, NW), lambda i: (i, 0)),
            pl.BlockSpec((1, HID), lambda i: (0, 0)),
            pl.BlockSpec((HID, HID), lambda i: (0, 0)),
        ],
        out_specs=pl.BlockSpec((_RB, HID), lambda i: (i, 0)),
        out_shape=jax.ShapeDtypeStruct((N_NODES, HID), jnp.float32),
    )(P, hs1, deg2, b1, W2)


def _tc3_body(q_ref, hs_ref, deg_ref, b2_ref, batch_ref,
              fc1w_ref, fc1b_ref, fc2w_ref, fc2b_ref, out_ref,
              acc_ref, cnt_ref):
    i = pl.program_id(0)

    @pl.when(i == 0)
    def _():
        acc_ref[...] = jnp.zeros_like(acc_ref)
        cnt_ref[...] = jnp.zeros_like(cnt_ref)

    dinv = _dinv_from(deg_ref)
    agg = dinv * (q_ref[0] + q_ref[1] + hs_ref[...]) + b2_ref[...]
    h = jnp.maximum(agg, 0.0)
    b = batch_ref[0]
    gids = lax.broadcasted_iota(jnp.int32, (NUM_GRAPHS, _RB), 0)
    onehot = (b == gids).astype(jnp.float32)
    acc_ref[...] += jnp.dot(onehot, h, preferred_element_type=jnp.float32)
    cnt_ref[...] += jnp.sum(onehot, axis=1, keepdims=True)

    @pl.when(i == _NBLK - 1)
    def _():
        pooled = acc_ref[...] / jnp.maximum(cnt_ref[...], 1.0)
        z = jnp.maximum(
            jnp.dot(pooled, fc1w_ref[...],
                    preferred_element_type=jnp.float32) + fc1b_ref[...], 0.0)
        out_ref[...] = jnp.dot(z, fc2w_ref[...],
                               preferred_element_type=jnp.float32) + fc2b_ref[...]


def _tc3(Q, hs2, deg2, b2, batch3, fc1_W, fc1_b, fc2_W, fc2_b):
    return pl.pallas_call(
        _tc3_body,
        grid=(_NBLK,),
        in_specs=[
            pl.BlockSpec((NC, _RB, HID), lambda i: (0, i, 0)),
            pl.BlockSpec((_RB, HID), lambda i: (i, 0)),
            pl.BlockSpec((_RB, NW), lambda i: (i, 0)),
            pl.BlockSpec((1, HID), lambda i: (0, 0)),
            pl.BlockSpec((1, 1, _RB), lambda i: (i, 0, 0)),
            pl.BlockSpec((HID, HID), lambda i: (0, 0)),
            pl.BlockSpec((1, HID), lambda i: (0, 0)),
            pl.BlockSpec((HID, OUT_DIM), lambda i: (0, 0)),
            pl.BlockSpec((1, OUT_DIM), lambda i: (0, 0)),
        ],
        out_specs=pl.BlockSpec((NUM_GRAPHS, OUT_DIM), lambda i: (0, 0)),
        out_shape=jax.ShapeDtypeStruct((NUM_GRAPHS, OUT_DIM), jnp.float32),
        scratch_shapes=[
            pltpu.VMEM((NUM_GRAPHS, HID), jnp.float32),
            pltpu.VMEM((NUM_GRAPHS, 1), jnp.float32),
        ],
    )(Q, hs2, deg2, b2, batch3, fc1_W, fc1_b, fc2_W, fc2_b)


def kernel(x, edge_index, batch, W1, b1, W2, b2, fc1_W, fc1_b, fc2_W, fc2_b):
    pad = NW * EPT - N_EDGES
    pad_src = (jnp.arange(pad, dtype=jnp.int32) * 13) % N_NODES
    pad_dst = N_NODES + (jnp.arange(pad, dtype=jnp.int32) % (NPAD - N_NODES))
    src3 = jnp.concatenate([edge_index[0], pad_src]).reshape(NW, NCH, K)
    dst3 = jnp.concatenate([edge_index[1], pad_dst]).reshape(NW, NCH, K)

    zerosH = jnp.zeros((NPAD, HID), jnp.float32)

    deg2 = _deg_call(dst3).T
    hs1 = _tc1(x, W1, deg2)
    P = _agg_call(src3, dst3, hs1, zerosH)
    hs2 = _tc2(P, hs1, deg2, b1.reshape(1, HID), W2)
    Q = _agg_call(src3, dst3, hs2, zerosH)
    out = _tc3(Q, hs2, deg2, b2.reshape(1, HID),
               batch.reshape(_NBLK, 1, _RB),
               fc1_W, fc1_b.reshape(1, HID), fc2_W, fc2_b.reshape(1, OUT_DIM))
    return out


# speedup vs baseline: 1.1001x; 1.1001x over previous
import functools

import jax
import jax.numpy as jnp
import numpy as np
from jax import lax
from jax.experimental import pallas as pl
from jax.experimental.pallas import tpu as pltpu
from jax.experimental.pallas import tpu_sc as plsc

N_NODES = 10000
N_EDGES = 320000
IN_DIM = 128
HID = 128
OUT_DIM = 64
NUM_GRAPHS = 64

NC, NS, L = 2, 16, 16
NW = NC * NS
K = 128
NCH = 80
EPT = K * NCH
NPAD = 10112
RPT = NPAD // NS
NCH_H = NCH // 2

_MESH = plsc.VectorSubcoreMesh(core_axis_name="c", subcore_axis_name="s",
                               num_cores=NC, num_subcores=NS)


def _deg_body(dst_hbm, out_hbm, idx_v, deg_v):
    c = lax.axis_index("c")
    s = lax.axis_index("s")
    wid = c * NS + s
    pltpu.sync_copy(dst_hbm.at[wid], idx_v)
    zero16 = jnp.zeros((L,), jnp.float32)
    ones16 = jnp.full((L,), 1.0, jnp.float32)

    def zbody(i, carry):
        deg_v[pl.ds(pl.multiple_of(i * L, L), L)] = zero16
        return carry

    lax.fori_loop(0, NPAD // L, zbody, 0)

    def body(j, carry):
        for ci in range(K // L):
            idx16 = idx_v[j, pl.ds(ci * L, L)]
            plsc.addupdate_scatter(deg_v, [idx16], ones16)
        return carry

    lax.fori_loop(0, NCH, body, 0)
    pltpu.sync_copy(deg_v, out_hbm.at[wid])


_deg_call = functools.partial(
    pl.kernel,
    out_type=jax.ShapeDtypeStruct((NW, NPAD), jnp.float32),
    mesh=_MESH,
    compiler_params=pltpu.CompilerParams(needs_layout_passes=False),
    scratch_types=[
        pltpu.VMEM((NCH, K), jnp.int32),
        pltpu.VMEM((NPAD,), jnp.float32),
    ],
)(_deg_body)


def _agg_body(src_hbm, dst_hbm, h_hbm, zeros_hbm, out_hbm,
              src_v, dst_v, rows_v, agg_sh, gsem0, gsem1):
    c = lax.axis_index("c")
    s = lax.axis_index("s")
    wid = c * NS + s
    r0 = pl.multiple_of(s * RPT, 8)

    pltpu.sync_copy(src_hbm.at[wid, pl.ds(0, NCH_H)], src_v)
    pltpu.sync_copy(dst_hbm.at[wid, pl.ds(0, NCH_H)], dst_v)
    pltpu.async_copy(h_hbm.at[src_v.at[0]], rows_v.at[0], gsem0)
    pltpu.sync_copy(zeros_hbm.at[pl.ds(r0, RPT)], agg_sh.at[pl.ds(r0, RPT)])
    plsc.subcore_barrier()

    def body(jj, carry):
        j0 = jj * 2
        j1 = j0 + 1
        pltpu.make_async_copy(h_hbm.at[src_v.at[j0]], rows_v.at[0], gsem0).wait()
        pltpu.async_copy(h_hbm.at[src_v.at[j1]], rows_v.at[1], gsem1)
        pltpu.sync_copy(rows_v.at[0], agg_sh.at[dst_v.at[j0]], add=True)
        pltpu.make_async_copy(h_hbm.at[src_v.at[j1]], rows_v.at[1], gsem1).wait()

        @pl.when(j1 + 1 < NCH_H)
        def _():
            pltpu.async_copy(h_hbm.at[src_v.at[j0 + 2]], rows_v.at[0], gsem0)

        pltpu.sync_copy(rows_v.at[1], agg_sh.at[dst_v.at[j1]], add=True)
        return carry

    lax.fori_loop(0, NCH_H // 2, body, 0)
    pltpu.sync_copy(src_hbm.at[wid, pl.ds(NCH_H, NCH_H)], src_v)
    pltpu.sync_copy(dst_hbm.at[wid, pl.ds(NCH_H, NCH_H)], dst_v)
    pltpu.async_copy(h_hbm.at[src_v.at[0]], rows_v.at[0], gsem0)
    lax.fori_loop(0, NCH_H // 2, body, 0)
    plsc.subcore_barrier()
    pltpu.sync_copy(agg_sh.at[pl.ds(r0, RPT)], out_hbm.at[c, pl.ds(r0, RPT)])


_agg_call = functools.partial(
    pl.kernel,
    out_type=jax.ShapeDtypeStruct((NC, NPAD, HID), jnp.float32),
    mesh=_MESH,
    scratch_types=[
        pltpu.VMEM((NCH_H, K), jnp.int32),
        pltpu.VMEM((NCH_H, K), jnp.int32),
        pltpu.VMEM((2, K, HID), jnp.float32),
        pltpu.VMEM_SHARED((NPAD, HID), jnp.float32),
        pltpu.SemaphoreType.DMA,
        pltpu.SemaphoreType.DMA,
    ],
)(_agg_body)


_RB = 1000
_NBLK = N_NODES // _RB


def _dinv_from(deg_ref):
    d = jnp.sum(deg_ref[...], axis=1, keepdims=True) + 1.0
    return lax.rsqrt(d)


def _tc1a_body(x_ref, w1_ref, out_ref):
    out_ref[...] = jnp.dot(x_ref[...], w1_ref[...],
                           preferred_element_type=jnp.float32)


def _tc1a(x, W1):
    return pl.pallas_call(
        _tc1a_body,
        grid=(_NBLK,),
        in_specs=[
            pl.BlockSpec((_RB, IN_DIM), lambda i: (i, 0)),
            pl.BlockSpec((IN_DIM, HID), lambda i: (0, 0)),
        ],
        out_specs=pl.BlockSpec((_RB, HID), lambda i: (i, 0)),
        out_shape=jax.ShapeDtypeStruct((N_NODES, HID), jnp.float32),
    )(x, W1)


def _tc1b_body(h_ref, deg_ref, out_ref):
    out_ref[...] = h_ref[...] * _dinv_from(deg_ref)


def _tc1b(h1, deg2):
    return pl.pallas_call(
        _tc1b_body,
        grid=(_NBLK,),
        in_specs=[
            pl.BlockSpec((_RB, HID), lambda i: (i, 0)),
            pl.BlockSpec((_RB, NW), lambda i: (i, 0)),
        ],
        out_specs=pl.BlockSpec((_RB, HID), lambda i: (i, 0)),
        out_shape=jax.ShapeDtypeStruct((N_NODES, HID), jnp.float32),
    )(h1, deg2)


def _tc2_body(p_ref, hs_ref, deg_ref, b1_ref, w2_ref, out_ref):
    dinv = _dinv_from(deg_ref)
    agg = dinv * (p_ref[0] + p_ref[1] + hs_ref[...]) + b1_ref[...]
    t = jnp.maximum(agg, 0.0)
    out_ref[...] = jnp.dot(t, w2_ref[...],
                           preferred_element_type=jnp.float32) * dinv


def _tc2(P, hs1, deg2, b1, W2):
    return pl.pallas_call(
        _tc2_body,
        grid=(_NBLK,),
        in_specs=[
            pl.BlockSpec((NC, _RB, HID), lambda i: (0, i, 0)),
            pl.BlockSpec((_RB, HID), lambda i: (i, 0)),
            pl.BlockSpec((_RB, NW), lambda i: (i, 0)),
            pl.BlockSpec((1, HID), lambda i: (0, 0)),
            pl.BlockSpec((HID, HID), lambda i: (0, 0)),
        ],
        out_specs=pl.BlockSpec((_RB, HID), lambda i: (i, 0)),
        out_shape=jax.ShapeDtypeStruct((N_NODES, HID), jnp.float32),
    )(P, hs1, deg2, b1, W2)


def _tc3_body(q_ref, hs_ref, deg_ref, b2_ref, batch_ref,
              fc1w_ref, fc1b_ref, fc2w_ref, fc2b_ref, out_ref,
              acc_ref, cnt_ref):
    i = pl.program_id(0)

    @pl.when(i == 0)
    def _():
        acc_ref[...] = jnp.zeros_like(acc_ref)
        cnt_ref[...] = jnp.zeros_like(cnt_ref)

    dinv = _dinv_from(deg_ref)
    agg = dinv * (q_ref[0] + q_ref[1] + hs_ref[...]) + b2_ref[...]
    h = jnp.maximum(agg, 0.0)
    b = batch_ref[0]
    gids = lax.broadcasted_iota(jnp.int32, (NUM_GRAPHS, _RB), 0)
    onehot = (b == gids).astype(jnp.float32)
    acc_ref[...] += jnp.dot(onehot, h, preferred_element_type=jnp.float32)
    cnt_ref[...] += jnp.sum(onehot, axis=1, keepdims=True)

    @pl.when(i == _NBLK - 1)
    def _():
        pooled = acc_ref[...] / jnp.maximum(cnt_ref[...], 1.0)
        z = jnp.maximum(
            jnp.dot(pooled, fc1w_ref[...],
                    preferred_element_type=jnp.float32) + fc1b_ref[...], 0.0)
        out_ref[...] = jnp.dot(z, fc2w_ref[...],
                               preferred_element_type=jnp.float32) + fc2b_ref[...]


def _tc3(Q, hs2, deg2, b2, batch3, fc1_W, fc1_b, fc2_W, fc2_b):
    return pl.pallas_call(
        _tc3_body,
        grid=(_NBLK,),
        in_specs=[
            pl.BlockSpec((NC, _RB, HID), lambda i: (0, i, 0)),
            pl.BlockSpec((_RB, HID), lambda i: (i, 0)),
            pl.BlockSpec((_RB, NW), lambda i: (i, 0)),
            pl.BlockSpec((1, HID), lambda i: (0, 0)),
            pl.BlockSpec((1, 1, _RB), lambda i: (i, 0, 0)),
            pl.BlockSpec((HID, HID), lambda i: (0, 0)),
            pl.BlockSpec((1, HID), lambda i: (0, 0)),
            pl.BlockSpec((HID, OUT_DIM), lambda i: (0, 0)),
            pl.BlockSpec((1, OUT_DIM), lambda i: (0, 0)),
        ],
        out_specs=pl.BlockSpec((NUM_GRAPHS, OUT_DIM), lambda i: (0, 0)),
        out_shape=jax.ShapeDtypeStruct((NUM_GRAPHS, OUT_DIM), jnp.float32),
        scratch_shapes=[
            pltpu.VMEM((NUM_GRAPHS, HID), jnp.float32),
            pltpu.VMEM((NUM_GRAPHS, 1), jnp.float32),
        ],
    )(Q, hs2, deg2, b2, batch3, fc1_W, fc1_b, fc2_W, fc2_b)


def kernel(x, edge_index, batch, W1, b1, W2, b2, fc1_W, fc1_b, fc2_W, fc2_b):
    pad = NW * EPT - N_EDGES
    pad_src = jnp.asarray((np.arange(pad, dtype=np.int32) * 13) % N_NODES)
    pad_dst = jnp.asarray(N_NODES + (np.arange(pad, dtype=np.int32) % (NPAD - N_NODES)))
    src3 = jnp.concatenate([edge_index[0], pad_src]).reshape(NW, NCH, K)
    dst3 = jnp.concatenate([edge_index[1], pad_dst]).reshape(NW, NCH, K)

    zerosH = jnp.zeros((NPAD, HID), jnp.float32)

    h1 = _tc1a(x, W1)
    deg2 = _deg_call(dst3).T
    hs1 = _tc1b(h1, deg2)
    P = _agg_call(src3, dst3, hs1, zerosH)
    hs2 = _tc2(P, hs1, deg2, b1.reshape(1, HID), W2)
    Q = _agg_call(src3, dst3, hs2, zerosH)
    out = _tc3(Q, hs2, deg2, b2.reshape(1, HID),
               batch.reshape(_NBLK, 1, _RB),
               fc1_W, fc1_b.reshape(1, HID), fc2_W, fc2_b.reshape(1, OUT_DIM))
    return out


# speedup vs baseline: 1.1055x; 1.0049x over previous
import functools

import jax
import jax.numpy as jnp
import numpy as np
from jax import lax
from jax.experimental import pallas as pl
from jax.experimental.pallas import tpu as pltpu
from jax.experimental.pallas import tpu_sc as plsc

N_NODES = 10000
N_EDGES = 320000
IN_DIM = 128
HID = 128
OUT_DIM = 64
NUM_GRAPHS = 64

NC, NS, L = 2, 16, 16
NW = NC * NS
K = 128
NCH = 80
EPT = K * NCH
NPAD = 10112
RPT = NPAD // NS
NCH_H = NCH // 2

_MESH = plsc.VectorSubcoreMesh(core_axis_name="c", subcore_axis_name="s",
                               num_cores=NC, num_subcores=NS)


def _deg_body(dst_hbm, out_hbm, idx_v, deg_v):
    c = lax.axis_index("c")
    s = lax.axis_index("s")
    wid = c * NS + s
    pltpu.sync_copy(dst_hbm.at[wid], idx_v)
    zero16 = jnp.zeros((L,), jnp.float32)
    ones16 = jnp.full((L,), 1.0, jnp.float32)

    def zbody(i, carry):
        deg_v[pl.ds(pl.multiple_of(i * L, L), L)] = zero16
        return carry

    lax.fori_loop(0, NPAD // L, zbody, 0)

    def body(j, carry):
        for ci in range(K // L):
            idx16 = idx_v[j, pl.ds(ci * L, L)]
            plsc.addupdate_scatter(deg_v, [idx16], ones16)
        return carry

    lax.fori_loop(0, NCH, body, 0)
    pltpu.sync_copy(deg_v, out_hbm.at[wid])


_deg_call = functools.partial(
    pl.kernel,
    out_type=jax.ShapeDtypeStruct((NW, NPAD), jnp.float32),
    mesh=_MESH,
    compiler_params=pltpu.CompilerParams(needs_layout_passes=False),
    scratch_types=[
        pltpu.VMEM((NCH, K), jnp.int32),
        pltpu.VMEM((NPAD,), jnp.float32),
    ],
)(_deg_body)


def _agg_body(src_hbm, dst_hbm, h_hbm, zeros_hbm, out_hbm,
              src_v, dst_v, rows_v, agg_sh, gsem0, gsem1):
    c = lax.axis_index("c")
    s = lax.axis_index("s")
    wid = c * NS + s
    r0 = pl.multiple_of(s * RPT, 8)

    pltpu.sync_copy(src_hbm.at[wid, pl.ds(0, NCH_H)], src_v)
    pltpu.sync_copy(dst_hbm.at[wid, pl.ds(0, NCH_H)], dst_v)
    pltpu.async_copy(h_hbm.at[src_v.at[0]], rows_v.at[0], gsem0)
    pltpu.sync_copy(zeros_hbm.at[pl.ds(r0, RPT)], agg_sh.at[pl.ds(r0, RPT)])
    plsc.subcore_barrier()

    def body(jj, carry):
        j0 = jj * 2
        j1 = j0 + 1
        pltpu.make_async_copy(h_hbm.at[src_v.at[j0]], rows_v.at[0], gsem0).wait()
        pltpu.async_copy(h_hbm.at[src_v.at[j1]], rows_v.at[1], gsem1)
        pltpu.sync_copy(rows_v.at[0], agg_sh.at[dst_v.at[j0]], add=True)
        pltpu.make_async_copy(h_hbm.at[src_v.at[j1]], rows_v.at[1], gsem1).wait()

        @pl.when(j1 + 1 < NCH_H)
        def _():
            pltpu.async_copy(h_hbm.at[src_v.at[j0 + 2]], rows_v.at[0], gsem0)

        pltpu.sync_copy(rows_v.at[1], agg_sh.at[dst_v.at[j1]], add=True)
        return carry

    lax.fori_loop(0, NCH_H // 2, body, 0)
    pltpu.sync_copy(src_hbm.at[wid, pl.ds(NCH_H, NCH_H)], src_v)
    pltpu.sync_copy(dst_hbm.at[wid, pl.ds(NCH_H, NCH_H)], dst_v)
    pltpu.async_copy(h_hbm.at[src_v.at[0]], rows_v.at[0], gsem0)
    lax.fori_loop(0, NCH_H // 2, body, 0)
    plsc.subcore_barrier()
    pltpu.sync_copy(agg_sh.at[pl.ds(r0, RPT)], out_hbm.at[c, pl.ds(r0, RPT)])


_agg_call = functools.partial(
    pl.kernel,
    out_type=jax.ShapeDtypeStruct((NC, NPAD, HID), jnp.float32),
    mesh=_MESH,
    scratch_types=[
        pltpu.VMEM((NCH_H, K), jnp.int32),
        pltpu.VMEM((NCH_H, K), jnp.int32),
        pltpu.VMEM((2, K, HID), jnp.float32),
        pltpu.VMEM_SHARED((NPAD, HID), jnp.float32),
        pltpu.SemaphoreType.DMA,
        pltpu.SemaphoreType.DMA,
    ],
)(_agg_body)


_RB = 1000
_NBLK = N_NODES // _RB


def _dinv_from(deg_ref):
    d = jnp.sum(deg_ref[...], axis=1, keepdims=True) + 1.0
    return lax.rsqrt(d)


def _tc1a_body(x_ref, w1_ref, out_ref):
    out_ref[...] = jnp.dot(x_ref[...], w1_ref[...],
                           preferred_element_type=jnp.float32)


def _tc1a(x, W1):
    return pl.pallas_call(
        _tc1a_body,
        grid=(_NBLK,),
        in_specs=[
            pl.BlockSpec((_RB, IN_DIM), lambda i: (i, 0)),
            pl.BlockSpec((IN_DIM, HID), lambda i: (0, 0)),
        ],
        out_specs=pl.BlockSpec((_RB, HID), lambda i: (i, 0)),
        out_shape=jax.ShapeDtypeStruct((N_NODES, HID), jnp.float32),
    )(x, W1)


def _tc1b_body(h_ref, deg_ref, out_ref):
    out_ref[...] = h_ref[...] * _dinv_from(deg_ref)


def _tc1b(h1, deg2):
    return pl.pallas_call(
        _tc1b_body,
        grid=(_NBLK,),
        in_specs=[
            pl.BlockSpec((_RB, HID), lambda i: (i, 0)),
            pl.BlockSpec((_RB, NW), lambda i: (i, 0)),
        ],
        out_specs=pl.BlockSpec((_RB, HID), lambda i: (i, 0)),
        out_shape=jax.ShapeDtypeStruct((N_NODES, HID), jnp.float32),
    )(h1, deg2)


def _tc2_body(p_ref, hs_ref, deg_ref, b1_ref, w2_ref, out_ref):
    dinv = _dinv_from(deg_ref)
    agg = dinv * (p_ref[0] + p_ref[1] + hs_ref[...]) + b1_ref[...]
    t = jnp.maximum(agg, 0.0)
    out_ref[...] = jnp.dot(t, w2_ref[...],
                           preferred_element_type=jnp.float32) * dinv


def _tc2(P, hs1, deg2, b1, W2):
    return pl.pallas_call(
        _tc2_body,
        grid=(_NBLK,),
        in_specs=[
            pl.BlockSpec((NC, _RB, HID), lambda i: (0, i, 0)),
            pl.BlockSpec((_RB, HID), lambda i: (i, 0)),
            pl.BlockSpec((_RB, NW), lambda i: (i, 0)),
            pl.BlockSpec((1, HID), lambda i: (0, 0)),
            pl.BlockSpec((HID, HID), lambda i: (0, 0)),
        ],
        out_specs=pl.BlockSpec((_RB, HID), lambda i: (i, 0)),
        out_shape=jax.ShapeDtypeStruct((N_NODES, HID), jnp.float32),
    )(P, hs1, deg2, b1, W2)


def _tc3_body(q_ref, hs_ref, deg_ref, b2_ref, batch_ref,
              fc1w_ref, fc1b_ref, fc2w_ref, fc2b_ref, out_ref,
              acc_ref, cnt_ref):
    i = pl.program_id(0)

    @pl.when(i == 0)
    def _():
        acc_ref[...] = jnp.zeros_like(acc_ref)
        cnt_ref[...] = jnp.zeros_like(cnt_ref)

    dinv = _dinv_from(deg_ref)
    agg = dinv * (q_ref[0] + q_ref[1] + hs_ref[...]) + b2_ref[...]
    h = jnp.maximum(agg, 0.0)
    b = batch_ref[0]
    gids = lax.broadcasted_iota(jnp.int32, (NUM_GRAPHS, _RB), 0)
    onehot = (b == gids).astype(jnp.float32)
    acc_ref[...] += jnp.dot(onehot, h, preferred_element_type=jnp.float32)
    cnt_ref[...] += jnp.sum(onehot, axis=1, keepdims=True)

    @pl.when(i == _NBLK - 1)
    def _():
        pooled = acc_ref[...] / jnp.maximum(cnt_ref[...], 1.0)
        z = jnp.maximum(
            jnp.dot(pooled, fc1w_ref[...],
                    preferred_element_type=jnp.float32) + fc1b_ref[...], 0.0)
        out_ref[...] = jnp.dot(z, fc2w_ref[...],
                               preferred_element_type=jnp.float32) + fc2b_ref[...]


def _tc3(Q, hs2, deg2, b2, batch3, fc1_W, fc1_b, fc2_W, fc2_b):
    return pl.pallas_call(
        _tc3_body,
        grid=(_NBLK,),
        in_specs=[
            pl.BlockSpec((NC, _RB, HID), lambda i: (0, i, 0)),
            pl.BlockSpec((_RB, HID), lambda i: (i, 0)),
            pl.BlockSpec((_RB, NW), lambda i: (i, 0)),
            pl.BlockSpec((1, HID), lambda i: (0, 0)),
            pl.BlockSpec((1, 1, _RB), lambda i: (i, 0, 0)),
            pl.BlockSpec((HID, HID), lambda i: (0, 0)),
            pl.BlockSpec((1, HID), lambda i: (0, 0)),
            pl.BlockSpec((HID, OUT_DIM), lambda i: (0, 0)),
            pl.BlockSpec((1, OUT_DIM), lambda i: (0, 0)),
        ],
        out_specs=pl.BlockSpec((NUM_GRAPHS, OUT_DIM), lambda i: (0, 0)),
        out_shape=jax.ShapeDtypeStruct((NUM_GRAPHS, OUT_DIM), jnp.float32),
        scratch_shapes=[
            pltpu.VMEM((NUM_GRAPHS, HID), jnp.float32),
            pltpu.VMEM((NUM_GRAPHS, 1), jnp.float32),
        ],
    )(Q, hs2, deg2, b2, batch3, fc1_W, fc1_b, fc2_W, fc2_b)


def kernel(x, edge_index, batch, W1, b1, W2, b2, fc1_W, fc1_b, fc2_W, fc2_b):
    pad = NW * EPT - N_EDGES
    pad_src = (np.arange(pad, dtype=np.int32) * 13) % N_NODES
    pad_dst = N_NODES + (np.arange(pad, dtype=np.int32) % (NPAD - N_NODES))
    pads = jnp.asarray(np.stack([pad_src, pad_dst]))
    e3 = jnp.concatenate([edge_index, pads], axis=1).reshape(2, NW, NCH, K)
    src3, dst3 = e3[0], e3[1]

    zerosH = jnp.zeros((NPAD, HID), jnp.float32)

    h1 = _tc1a(x, W1)
    deg2 = _deg_call(dst3).T
    hs1 = _tc1b(h1, deg2)
    P = _agg_call(src3, dst3, hs1, zerosH)
    hs2 = _tc2(P, hs1, deg2, b1.reshape(1, HID), W2)
    Q = _agg_call(src3, dst3, hs2, zerosH)
    out = _tc3(Q, hs2, deg2, b2.reshape(1, HID),
               batch.reshape(_NBLK, 1, _RB),
               fc1_W, fc1_b.reshape(1, HID), fc2_W, fc2_b.reshape(1, OUT_DIM))
    return out
